# Initial kernel scaffold; baseline (speedup 1.0000x reference)
import jax
import jax.numpy as jnp
from jax.experimental import pallas as pl


def kernel(x, edge_index, batch, W_rel1, b_rel1, W_root1, W_rel2, b_rel2, W_root2, W_rel3, b_rel3, W_root3, W_mp1, b_mp1, W_mp2, b_mp2):
    raise NotImplementedError("write your pallas kernel here")


# speedup vs baseline: 5.1273x; 5.1273x over previous
import functools

import jax
import jax.numpy as jnp
from jax import lax
from jax.experimental import pallas as pl
from jax.experimental.pallas import tpu as pltpu
from jax.experimental.pallas import tpu_sc as plsc

N = 10000
E = 320000
DIN = 128
DH = 256
DOUT = 128
G = 16

NC = 2
NS = 16
EPW = E // NS
CH = 128
NFULL = EPW // CH
REM = EPW - NFULL * CH
NPAD = 10240
NT = NPAD // NS

RB = 1000
NRB = N // RB


def _make_sc_segsum(D2):
    mesh = plsc.VectorSubcoreMesh(
        core_axis_name="c", subcore_axis_name="s", num_cores=NC, num_subcores=NS
    )

    @functools.partial(
        pl.kernel,
        out_type=jax.ShapeDtypeStruct((NC, NPAD, D2), jnp.float32),
        mesh=mesh,
        scratch_types=[
            pltpu.VMEM((CH,), jnp.int32),
            pltpu.VMEM((CH,), jnp.int32),
            pltpu.VMEM((CH, D2), jnp.float32),
            pltpu.VMEM((REM,), jnp.int32),
            pltpu.VMEM((REM,), jnp.int32),
            pltpu.VMEM((REM, D2), jnp.float32),
            pltpu.VMEM_SHARED((NPAD, D2), jnp.float32),
            pltpu.SemaphoreType.DMA,
        ],
    )
    def seg(h_hbm, src_hbm, dst_hbm, out_hbm, srcv, dstv, rows, srcr, dstr,
            rowsr, acc, sem):
        c = lax.axis_index("c")
        s = lax.axis_index("s")

        def _zrow(i, _):
            for jj in range(D2 // 16):
                rows[i, pl.ds(jj * 16, 16)] = jnp.zeros((16,), jnp.float32)
            return 0

        lax.fori_loop(0, CH, _zrow, 0)
        zbase = pl.multiple_of(s * NT, 8)
        for k in range(NT // CH):
            pltpu.sync_copy(rows, acc.at[pl.ds(zbase + k * CH, CH)])
        plsc.subcore_barrier()

        ebase = s * EPW

        def _chunk(j, _):
            off = pl.multiple_of(ebase + j * CH, 8)
            pltpu.sync_copy(src_hbm.at[pl.ds(off, CH)], srcv)
            pltpu.sync_copy(dst_hbm.at[pl.ds(off, CH)], dstv)
            pltpu.async_copy(h_hbm.at[c].at[srcv], rows, sem).wait()
            pltpu.sync_copy(rows, acc.at[dstv], add=True)
            return 0

        lax.fori_loop(0, NFULL, _chunk, 0)

        if REM:
            roff = pl.multiple_of(ebase + NFULL * CH, 8)
            pltpu.sync_copy(src_hbm.at[pl.ds(roff, REM)], srcr)
            pltpu.sync_copy(dst_hbm.at[pl.ds(roff, REM)], dstr)
            pltpu.async_copy(h_hbm.at[c].at[srcr], rowsr, sem).wait()
            pltpu.sync_copy(rowsr, acc.at[dstr], add=True)

        plsc.subcore_barrier()
        wbase = pl.multiple_of(s * NT, 8)
        pltpu.sync_copy(acc.at[pl.ds(wbase, NT)],
                        out_hbm.at[c].at[pl.ds(wbase, NT)])

    return seg


EPW1 = E // (NC * NS)
NFULL1 = EPW1 // CH
REM1 = EPW1 - NFULL1 * CH


def _make_sc_segsum_edgesplit():
    D2 = DIN
    mesh = plsc.VectorSubcoreMesh(
        core_axis_name="c", subcore_axis_name="s", num_cores=NC, num_subcores=NS
    )

    @functools.partial(
        pl.kernel,
        out_type=jax.ShapeDtypeStruct((NC, NPAD, D2), jnp.float32),
        mesh=mesh,
        scratch_types=[
            pltpu.VMEM((CH,), jnp.int32),
            pltpu.VMEM((CH,), jnp.int32),
            pltpu.VMEM((CH, D2), jnp.float32),
            pltpu.VMEM((REM1,), jnp.int32),
            pltpu.VMEM((REM1,), jnp.int32),
            pltpu.VMEM((REM1, D2), jnp.float32),
            pltpu.VMEM_SHARED((NPAD, D2), jnp.float32),
            pltpu.SemaphoreType.DMA,
        ],
    )
    def seg(h_hbm, src_hbm, dst_hbm, out_hbm, srcv, dstv, rows, srcr, dstr,
            rowsr, acc, sem):
        c = lax.axis_index("c")
        s = lax.axis_index("s")

        def _zrow(i, _):
            for jj in range(D2 // 16):
                rows[i, pl.ds(jj * 16, 16)] = jnp.zeros((16,), jnp.float32)
            return 0

        lax.fori_loop(0, CH, _zrow, 0)
        zbase = pl.multiple_of(s * NT, 8)
        for k in range(NT // CH):
            pltpu.sync_copy(rows, acc.at[pl.ds(zbase + k * CH, CH)])
        plsc.subcore_barrier()

        ebase = (c * NS + s) * EPW1

        def _chunk(j, _):
            off = pl.multiple_of(ebase + j * CH, 8)
            pltpu.sync_copy(src_hbm.at[pl.ds(off, CH)], srcv)
            pltpu.sync_copy(dst_hbm.at[pl.ds(off, CH)], dstv)
            pltpu.async_copy(h_hbm.at[srcv], rows, sem).wait()
            pltpu.sync_copy(rows, acc.at[dstv], add=True)
            return 0

        lax.fori_loop(0, NFULL1, _chunk, 0)

        if REM1:
            roff = pl.multiple_of(ebase + NFULL1 * CH, 8)
            pltpu.sync_copy(src_hbm.at[pl.ds(roff, REM1)], srcr)
            pltpu.sync_copy(dst_hbm.at[pl.ds(roff, REM1)], dstr)
            pltpu.async_copy(h_hbm.at[srcr], rowsr, sem).wait()
            pltpu.sync_copy(rowsr, acc.at[dstr], add=True)

        plsc.subcore_barrier()
        wbase = pl.multiple_of(s * NT, 8)
        pltpu.sync_copy(acc.at[pl.ds(wbase, NT)],
                        out_hbm.at[c].at[pl.ds(wbase, NT)])

    return seg


def _tc_layer1_body(agg_ref, x_ref, wrel_ref, wroot_ref, b_ref, out_ref):
    agg = agg_ref[0] + agg_ref[1]
    acc = jnp.dot(agg, wrel_ref[...], preferred_element_type=jnp.float32)
    acc += jnp.dot(x_ref[...], wroot_ref[...], preferred_element_type=jnp.float32)
    acc += b_ref[...]
    h = jnp.maximum(acc, 0.0)
    out_ref[0] = h[:, :DH // 2]
    out_ref[1] = h[:, DH // 2:]


def _tc_layer1(agg3, x, wrel, wroot, b2):
    return pl.pallas_call(
        _tc_layer1_body,
        grid=(NRB,),
        in_specs=[
            pl.BlockSpec((NC, RB, DIN), lambda i: (0, i, 0)),
            pl.BlockSpec((RB, DIN), lambda i: (i, 0)),
            pl.BlockSpec(wrel.shape, lambda i: (0, 0)),
            pl.BlockSpec(wroot.shape, lambda i: (0, 0)),
            pl.BlockSpec(b2.shape, lambda i: (0, 0)),
        ],
        out_specs=pl.BlockSpec((NC, RB, DH // 2), lambda i: (0, i, 0)),
        out_shape=jax.ShapeDtypeStruct((NC, N, DH // 2), jnp.float32),
    )(agg3, x, wrel, wroot, b2)


def _tc_layer_body(agg_ref, x_ref, wrel_ref, wroot_ref, b_ref, out_ref):
    acc = jnp.dot(agg_ref[0], wrel_ref[0], preferred_element_type=jnp.float32)
    acc += jnp.dot(agg_ref[1], wrel_ref[1], preferred_element_type=jnp.float32)
    acc += jnp.dot(x_ref[0], wroot_ref[0], preferred_element_type=jnp.float32)
    acc += jnp.dot(x_ref[1], wroot_ref[1], preferred_element_type=jnp.float32)
    acc += b_ref[...]
    h = jnp.maximum(acc, 0.0)
    out_ref[0] = h[:, :DH // 2]
    out_ref[1] = h[:, DH // 2:]


def _tc_layer(agg3, x3, wrel2, wroot2, b2):
    d2 = agg3.shape[2]
    return pl.pallas_call(
        _tc_layer_body,
        grid=(NRB,),
        in_specs=[
            pl.BlockSpec((NC, RB, d2), lambda i: (0, i, 0)),
            pl.BlockSpec((NC, RB, d2), lambda i: (0, i, 0)),
            pl.BlockSpec(wrel2.shape, lambda i: (0, 0, 0)),
            pl.BlockSpec(wroot2.shape, lambda i: (0, 0, 0)),
            pl.BlockSpec(b2.shape, lambda i: (0, 0)),
        ],
        out_specs=pl.BlockSpec((NC, RB, DH // 2), lambda i: (0, i, 0)),
        out_shape=jax.ShapeDtypeStruct((NC, N, DH // 2), jnp.float32),
    )(agg3, x3, wrel2, wroot2, b2)


def _tc_layer3_body(agg_ref, x_ref, wrel_ref, wroot_ref, b_ref, batch_ref,
                    emb_ref, psum_ref, pcnt_ref):
    i = pl.program_id(0)
    acc = jnp.dot(agg_ref[0], wrel_ref[0], preferred_element_type=jnp.float32)
    acc += jnp.dot(agg_ref[1], wrel_ref[1], preferred_element_type=jnp.float32)
    acc += jnp.dot(x_ref[0], wroot_ref[0], preferred_element_type=jnp.float32)
    acc += jnp.dot(x_ref[1], wroot_ref[1], preferred_element_type=jnp.float32)
    acc += b_ref[...]
    emb_ref[...] = acc
    h = jnp.maximum(acc, 0.0)

    bvec = batch_ref[0]
    gids = lax.broadcasted_iota(jnp.int32, (G, RB), 0)
    onehot = jnp.where(bvec == gids, 1.0, 0.0)

    @pl.when(i == 0)
    def _():
        psum_ref[...] = jnp.zeros_like(psum_ref)
        pcnt_ref[...] = jnp.zeros_like(pcnt_ref)

    psum_ref[...] += jnp.dot(onehot, h, preferred_element_type=jnp.float32)
    pcnt_ref[...] += jnp.dot(
        onehot, jnp.ones((RB, 128), jnp.float32),
        preferred_element_type=jnp.float32)


def _tc_layer3(agg3, x3, wrel2, wroot2, b2, batch3):
    return pl.pallas_call(
        _tc_layer3_body,
        grid=(NRB,),
        in_specs=[
            pl.BlockSpec((NC, RB, DH // 2), lambda i: (0, i, 0)),
            pl.BlockSpec((NC, RB, DH // 2), lambda i: (0, i, 0)),
            pl.BlockSpec(wrel2.shape, lambda i: (0, 0, 0)),
            pl.BlockSpec(wroot2.shape, lambda i: (0, 0, 0)),
            pl.BlockSpec(b2.shape, lambda i: (0, 0)),
            pl.BlockSpec((1, 1, RB), lambda i: (i, 0, 0)),
        ],
        out_specs=[
            pl.BlockSpec((RB, DH), lambda i: (i, 0)),
            pl.BlockSpec((G, DH), lambda i: (0, 0)),
            pl.BlockSpec((G, 128), lambda i: (0, 0)),
        ],
        out_shape=[
            jax.ShapeDtypeStruct((N, DH), jnp.float32),
            jax.ShapeDtypeStruct((G, DH), jnp.float32),
            jax.ShapeDtypeStruct((G, 128), jnp.float32),
        ],
    )(agg3, x3, wrel2, wroot2, b2, batch3)


def _tc_mlp_body(psum_ref, pcnt_ref, w1_ref, b1_ref, w2_ref, b2_ref, out_ref):
    cnt = jnp.maximum(pcnt_ref[:, :1], 1.0)
    pooled = psum_ref[...] / cnt
    h = jnp.dot(pooled, w1_ref[...], preferred_element_type=jnp.float32)
    h += b1_ref[...]
    o = jnp.dot(h, w2_ref[...], preferred_element_type=jnp.float32)
    o += b2_ref[...]
    out_ref[...] = o


def _tc_mlp(psum, pcnt, w1, b1, w2, b2):
    return pl.pallas_call(
        _tc_mlp_body,
        out_shape=jax.ShapeDtypeStruct((G, DOUT), jnp.float32),
    )(psum, pcnt, w1, b1, w2, b2)


def kernel(x, edge_index, batch, W_rel1, b_rel1, W_root1, W_rel2, b_rel2,
           W_root2, W_rel3, b_rel3, W_root3, W_mp1, b_mp1, W_mp2, b_mp2):
    src = edge_index[0]
    dst = edge_index[1]

    batch3 = batch.reshape(NRB, 1, RB)

    wrel2 = W_rel2.reshape(NC, DH // NC, DH)
    wroot2 = W_root2.reshape(NC, DH // NC, DH)
    wrel3 = W_rel3.reshape(NC, DH // NC, DH)
    wroot3 = W_root3.reshape(NC, DH // NC, DH)
    b1 = b_rel1.reshape(1, DH)
    b2 = b_rel2.reshape(1, DH)
    b3 = b_rel3.reshape(1, DH)
    bm1 = b_mp1.reshape(1, DH)
    bm2 = b_mp2.reshape(1, DOUT)

    seg1 = _make_sc_segsum_edgesplit()
    seg128 = _make_sc_segsum(DH // NC)

    agg1 = seg1(x, src, dst)
    h1 = _tc_layer1(agg1, x, W_rel1, W_root1, b1)
    agg2 = seg128(h1, src, dst)
    h2 = _tc_layer(agg2, h1, wrel2, wroot2, b2)
    agg3 = seg128(h2, src, dst)
    emb, psum, pcnt = _tc_layer3(agg3, h2, wrel3, wroot3, b3, batch3)
    out = _tc_mlp(psum, pcnt, W_mp1, bm1, W_mp2, bm2)
    return (emb, out)
